# baseline (speedup 1.0000x reference)
import functools

import jax
import jax.numpy as jnp
from jax import lax
from jax.experimental import pallas as pl
from jax.experimental.pallas import tpu as pltpu
from jax.experimental.pallas import tpu_sc as plsc

NC = 1
NS = 8
NW = NC * NS
L = 16

ROWS = 128
COLS = 32768
TC_ROWS = 96
SC_ROWS = ROWS - TC_ROWS
ROWS_PER_W = SC_ROWS // NW
GV = 8
GELEMS = GV * L
GROUPS = COLS // GELEMS
UNROLL_G = 4
ITERS = GROUPS // UNROLL_G

TC_BR = 8

NEG_INF = float("-inf")


def _tournament(ks, is_):
    while len(ks) > 1:
        nk, ni = [], []
        for a in range(0, len(ks), 2):
            k0, i0 = ks[a], is_[a]
            k1, i1 = ks[a + 1], is_[a + 1]
            better = (k1 > k0) | ((k1 == k0) & (i1 < i0))
            nk.append(jnp.where(better, k1, k0))
            ni.append(jnp.where(better, i1, i0))
        ks, is_ = nk, ni
    return ks[0], is_[0]


@functools.partial(
    pl.kernel,
    out_type=jax.ShapeDtypeStruct((NW, L), jnp.int32),
    mesh=plsc.VectorSubcoreMesh(
        core_axis_name="c", subcore_axis_name="s", num_cores=NC,
        num_subcores=NS,
    ),
    scratch_types=[
        pltpu.VMEM((2, COLS), jnp.float32),
        pltpu.VMEM((L,), jnp.int32),
        pltpu.SemaphoreType.DMA,
        pltpu.SemaphoreType.DMA,
    ],
)
def _argmax_sc(logits_hbm, out_hbm, buf, res, sem0, sem1):
    wid = lax.axis_index("s") * NC + lax.axis_index("c")
    sems = (sem0, sem1)
    lane = lax.iota(jnp.int32, L)

    def start_dma(r, slot):
        return pltpu.async_copy(
            logits_hbm.at[TC_ROWS + wid * ROWS_PER_W + r], buf.at[slot], sems[slot]
        )

    cps = [None, None]
    cps[0] = start_dma(0, 0)
    res_vec = jnp.zeros((L,), jnp.int32)

    for r in range(ROWS_PER_W):
        slot = r & 1
        if r + 1 < ROWS_PER_W:
            cps[1 - slot] = start_dma(r + 1, 1 - slot)
        cps[slot].wait()

        def body(k, carry):
            m, bb = carry
            g0 = k * UNROLL_G
            for u in range(UNROLL_G):
                base = (g0 + u) * GELEMS
                v = [buf[slot, pl.ds(base + j * L, L)] for j in range(GV)]
                while len(v) > 1:
                    v = [jnp.maximum(v[a], v[a + 1]) for a in range(0, len(v), 2)]
                pred = v[0] > m
                m = jnp.where(pred, v[0], m)
                bb = jnp.where(pred, g0 + u, bb)
            return m, bb

        m0 = jnp.full((L,), NEG_INF, jnp.float32)
        b0 = jnp.zeros((L,), jnp.int32)
        m, bb = lax.fori_loop(0, ITERS, body, (m0, b0))

        _, gstar = _tournament(
            [m[j] for j in range(L)], [bb[j] for j in range(L)]
        )

        gbase = gstar * GELEMS
        ivbase = gbase + lane
        fm = jnp.full((L,), NEG_INF, jnp.float32)
        fi = jnp.zeros((L,), jnp.int32)
        for j in range(GV):
            v = buf[slot, pl.ds(gbase + j * L, L)]
            pred = v > fm
            fm = jnp.where(pred, v, fm)
            fi = jnp.where(pred, ivbase + j * L, fi)
        _, ridx = _tournament([fm[j] for j in range(L)], [fi[j] for j in range(L)])
        res_vec = jnp.where(lane == r, ridx, res_vec)

    res[...] = res_vec
    pltpu.sync_copy(res, out_hbm.at[wid])


def _tc_body(x_ref, o_ref):
    x = x_ref[...]
    mx = jnp.max(x, axis=1, keepdims=True)
    idx = lax.broadcasted_iota(jnp.int32, x.shape, 1)
    cand = jnp.where(x == mx, idx, COLS)
    o_ref[...] = jnp.min(cand, axis=1, keepdims=True)


_argmax_tc = pl.pallas_call(
    _tc_body,
    grid=(TC_ROWS // TC_BR,),
    in_specs=[pl.BlockSpec((TC_BR, COLS), lambda i: (i, 0))],
    out_specs=pl.BlockSpec((TC_BR, 1), lambda i: (i, 0)),
    out_shape=jax.ShapeDtypeStruct((TC_ROWS, 1), jnp.int32),
)


def kernel(logits):
    tc_idx = _argmax_tc(logits)
    staged = _argmax_sc(logits)
    sc_idx = staged[:, :ROWS_PER_W].reshape(SC_ROWS, 1)
    return jnp.concatenate([tc_idx, sc_idx], axis=0)


# speedup vs baseline: 1.0201x; 1.0201x over previous
import functools

import jax
import jax.numpy as jnp
from jax import lax
from jax.experimental import pallas as pl
from jax.experimental.pallas import tpu as pltpu
from jax.experimental.pallas import tpu_sc as plsc

NC = 1
NS = 16
NW = NC * NS
L = 16

ROWS = 128
COLS = 32768
TC_ROWS = 112
SC_ROWS = ROWS - TC_ROWS
ROWS_PER_W = SC_ROWS // NW
GV = 8
GELEMS = GV * L
GROUPS = COLS // GELEMS
UNROLL_G = 4
ITERS = GROUPS // UNROLL_G

TC_BR = 8

NEG_INF = float("-inf")


def _tournament(ks, is_):
    while len(ks) > 1:
        nk, ni = [], []
        for a in range(0, len(ks), 2):
            k0, i0 = ks[a], is_[a]
            k1, i1 = ks[a + 1], is_[a + 1]
            better = (k1 > k0) | ((k1 == k0) & (i1 < i0))
            nk.append(jnp.where(better, k1, k0))
            ni.append(jnp.where(better, i1, i0))
        ks, is_ = nk, ni
    return ks[0], is_[0]


@functools.partial(
    pl.kernel,
    out_type=jax.ShapeDtypeStruct((NW, L), jnp.int32),
    mesh=plsc.VectorSubcoreMesh(
        core_axis_name="c", subcore_axis_name="s", num_cores=NC,
        num_subcores=NS,
    ),
    scratch_types=[
        pltpu.VMEM((2, COLS), jnp.float32),
        pltpu.VMEM((L,), jnp.int32),
        pltpu.SemaphoreType.DMA,
        pltpu.SemaphoreType.DMA,
    ],
)
def _argmax_sc(logits_hbm, out_hbm, buf, res, sem0, sem1):
    wid = lax.axis_index("s") * NC + lax.axis_index("c")
    sems = (sem0, sem1)
    lane = lax.iota(jnp.int32, L)

    def start_dma(r, slot):
        return pltpu.async_copy(
            logits_hbm.at[TC_ROWS + wid * ROWS_PER_W + r], buf.at[slot], sems[slot]
        )

    cps = [None, None]
    cps[0] = start_dma(0, 0)
    res_vec = jnp.zeros((L,), jnp.int32)

    for r in range(ROWS_PER_W):
        slot = r & 1
        if r + 1 < ROWS_PER_W:
            cps[1 - slot] = start_dma(r + 1, 1 - slot)
        cps[slot].wait()

        def body(k, carry):
            m, bb = carry
            g0 = k * UNROLL_G
            for u in range(UNROLL_G):
                base = (g0 + u) * GELEMS
                v = [buf[slot, pl.ds(base + j * L, L)] for j in range(GV)]
                while len(v) > 1:
                    v = [jnp.maximum(v[a], v[a + 1]) for a in range(0, len(v), 2)]
                pred = v[0] > m
                m = jnp.where(pred, v[0], m)
                bb = jnp.where(pred, g0 + u, bb)
            return m, bb

        m0 = jnp.full((L,), NEG_INF, jnp.float32)
        b0 = jnp.zeros((L,), jnp.int32)
        m, bb = lax.fori_loop(0, ITERS, body, (m0, b0))

        _, gstar = _tournament(
            [m[j] for j in range(L)], [bb[j] for j in range(L)]
        )

        gbase = gstar * GELEMS
        ivbase = gbase + lane
        fm = jnp.full((L,), NEG_INF, jnp.float32)
        fi = jnp.zeros((L,), jnp.int32)
        for j in range(GV):
            v = buf[slot, pl.ds(gbase + j * L, L)]
            pred = v > fm
            fm = jnp.where(pred, v, fm)
            fi = jnp.where(pred, ivbase + j * L, fi)
        _, ridx = _tournament([fm[j] for j in range(L)], [fi[j] for j in range(L)])
        res_vec = jnp.where(lane == r, ridx, res_vec)

    res[...] = res_vec
    pltpu.sync_copy(res, out_hbm.at[wid])


def _tc_body(x_ref, o_ref):
    x = x_ref[...]
    mx = jnp.max(x, axis=1, keepdims=True)
    idx = lax.broadcasted_iota(jnp.int32, x.shape, 1)
    cand = jnp.where(x == mx, idx, COLS)
    o_ref[...] = jnp.min(cand, axis=1, keepdims=True)


_argmax_tc = pl.pallas_call(
    _tc_body,
    grid=(TC_ROWS // TC_BR,),
    in_specs=[pl.BlockSpec((TC_BR, COLS), lambda i: (i, 0))],
    out_specs=pl.BlockSpec((TC_BR, 1), lambda i: (i, 0)),
    out_shape=jax.ShapeDtypeStruct((TC_ROWS, 1), jnp.int32),
)


def kernel(logits):
    tc_idx = _argmax_tc(logits)
    staged = _argmax_sc(logits)
    sc_idx = staged[:, :ROWS_PER_W].reshape(SC_ROWS, 1)
    return jnp.concatenate([tc_idx, sc_idx], axis=0)


# speedup vs baseline: 1.0754x; 1.0542x over previous
import functools

import jax
import jax.numpy as jnp
from jax import lax
from jax.experimental import pallas as pl
from jax.experimental.pallas import tpu as pltpu
from jax.experimental.pallas import tpu_sc as plsc

NC = 1
NS = 16
NW = NC * NS
L = 16

ROWS = 128
COLS = 32768
TC_ROWS = 96
SC_ROWS = ROWS - TC_ROWS
ROWS_PER_W = SC_ROWS // NW
GV = 8
GELEMS = GV * L
GROUPS = COLS // GELEMS
UNROLL_G = 4
ITERS = GROUPS // UNROLL_G

TC_BR = 8

NEG_INF = float("-inf")


def _tournament(ks, is_):
    while len(ks) > 1:
        nk, ni = [], []
        for a in range(0, len(ks), 2):
            k0, i0 = ks[a], is_[a]
            k1, i1 = ks[a + 1], is_[a + 1]
            better = (k1 > k0) | ((k1 == k0) & (i1 < i0))
            nk.append(jnp.where(better, k1, k0))
            ni.append(jnp.where(better, i1, i0))
        ks, is_ = nk, ni
    return ks[0], is_[0]


@functools.partial(
    pl.kernel,
    out_type=jax.ShapeDtypeStruct((NW, L), jnp.int32),
    mesh=plsc.VectorSubcoreMesh(
        core_axis_name="c", subcore_axis_name="s", num_cores=NC,
        num_subcores=NS,
    ),
    scratch_types=[
        pltpu.VMEM((2, COLS), jnp.float32),
        pltpu.VMEM((L,), jnp.int32),
        pltpu.SemaphoreType.DMA,
        pltpu.SemaphoreType.DMA,
    ],
)
def _argmax_sc(logits_hbm, out_hbm, buf, res, sem0, sem1):
    wid = lax.axis_index("s") * NC + lax.axis_index("c")
    sems = (sem0, sem1)
    lane = lax.iota(jnp.int32, L)

    def start_dma(r, slot):
        return pltpu.async_copy(
            logits_hbm.at[TC_ROWS + wid * ROWS_PER_W + r], buf.at[slot], sems[slot]
        )

    cps = [None, None]
    cps[0] = start_dma(0, 0)
    res_vec = jnp.zeros((L,), jnp.int32)

    for r in range(ROWS_PER_W):
        slot = r & 1
        if r + 1 < ROWS_PER_W:
            cps[1 - slot] = start_dma(r + 1, 1 - slot)
        cps[slot].wait()

        def body(k, carry):
            m, bb = carry
            g0 = k * UNROLL_G
            for u in range(UNROLL_G):
                base = (g0 + u) * GELEMS
                v = [buf[slot, pl.ds(base + j * L, L)] for j in range(GV)]
                while len(v) > 1:
                    v = [jnp.maximum(v[a], v[a + 1]) for a in range(0, len(v), 2)]
                pred = v[0] > m
                m = jnp.where(pred, v[0], m)
                bb = jnp.where(pred, g0 + u, bb)
            return m, bb

        m0 = jnp.full((L,), NEG_INF, jnp.float32)
        b0 = jnp.zeros((L,), jnp.int32)
        m, bb = lax.fori_loop(0, ITERS, body, (m0, b0))

        _, gstar = _tournament(
            [m[j] for j in range(L)], [bb[j] for j in range(L)]
        )

        gbase = gstar * GELEMS
        ivbase = gbase + lane
        fm = jnp.full((L,), NEG_INF, jnp.float32)
        fi = jnp.zeros((L,), jnp.int32)
        for j in range(GV):
            v = buf[slot, pl.ds(gbase + j * L, L)]
            pred = v > fm
            fm = jnp.where(pred, v, fm)
            fi = jnp.where(pred, ivbase + j * L, fi)
        _, ridx = _tournament([fm[j] for j in range(L)], [fi[j] for j in range(L)])
        res_vec = jnp.where(lane == r, ridx, res_vec)

    res[...] = res_vec
    pltpu.sync_copy(res, out_hbm.at[wid])


def _tc_body(x_ref, o_ref):
    x = x_ref[...]
    mx = jnp.max(x, axis=1, keepdims=True)
    idx = lax.broadcasted_iota(jnp.int32, x.shape, 1)
    cand = jnp.where(x == mx, idx, COLS)
    o_ref[...] = jnp.min(cand, axis=1, keepdims=True)


_argmax_tc = pl.pallas_call(
    _tc_body,
    grid=(TC_ROWS // TC_BR,),
    in_specs=[pl.BlockSpec((TC_BR, COLS), lambda i: (i, 0))],
    out_specs=pl.BlockSpec((TC_BR, 1), lambda i: (i, 0)),
    out_shape=jax.ShapeDtypeStruct((TC_ROWS, 1), jnp.int32),
)


def kernel(logits):
    tc_idx = _argmax_tc(logits)
    staged = _argmax_sc(logits)
    sc_idx = staged[:, :ROWS_PER_W].reshape(SC_ROWS, 1)
    return jnp.concatenate([tc_idx, sc_idx], axis=0)


# speedup vs baseline: 1.0809x; 1.0051x over previous
import functools

import jax
import jax.numpy as jnp
from jax import lax
from jax.experimental import pallas as pl
from jax.experimental.pallas import tpu as pltpu
from jax.experimental.pallas import tpu_sc as plsc

NC = 1
NS = 16
NW = NC * NS
L = 16

ROWS = 128
COLS = 32768
TC_ROWS = 96
SC_ROWS = ROWS - TC_ROWS
ROWS_PER_W = SC_ROWS // NW
GV = 8
GELEMS = GV * L
GROUPS = COLS // GELEMS
UNROLL_G = 4
ITERS = GROUPS // UNROLL_G

TC_BR = 8

NEG_INF = float("-inf")


def _tournament(ks, is_):
    while len(ks) > 1:
        nk, ni = [], []
        for a in range(0, len(ks), 2):
            k0, i0 = ks[a], is_[a]
            k1, i1 = ks[a + 1], is_[a + 1]
            better = (k1 > k0) | ((k1 == k0) & (i1 < i0))
            nk.append(jnp.where(better, k1, k0))
            ni.append(jnp.where(better, i1, i0))
        ks, is_ = nk, ni
    return ks[0], is_[0]


@functools.partial(
    pl.kernel,
    out_type=jax.ShapeDtypeStruct((NW, L), jnp.int32),
    mesh=plsc.VectorSubcoreMesh(
        core_axis_name="c", subcore_axis_name="s", num_cores=NC,
        num_subcores=NS,
    ),
    scratch_types=[
        pltpu.VMEM((2, COLS), jnp.float32),
        pltpu.VMEM((L,), jnp.int32),
        pltpu.SemaphoreType.DMA,
        pltpu.SemaphoreType.DMA,
    ],
)
def _argmax_sc(logits_hbm, out_hbm, buf, res, sem0, sem1):
    wid = lax.axis_index("s") * NC + lax.axis_index("c")
    sems = (sem0, sem1)
    lane = lax.iota(jnp.int32, L)

    def start_dma(r, slot):
        return pltpu.async_copy(
            logits_hbm.at[TC_ROWS + wid * ROWS_PER_W + r], buf.at[slot], sems[slot]
        )

    cps = [None, None]
    cps[0] = start_dma(0, 0)
    res_vec = jnp.zeros((L,), jnp.int32)

    for r in range(ROWS_PER_W):
        slot = r & 1
        if r + 1 < ROWS_PER_W:
            cps[1 - slot] = start_dma(r + 1, 1 - slot)
        cps[slot].wait()

        m0 = jnp.full((L,), NEG_INF, jnp.float32)
        b0 = jnp.zeros((L,), jnp.int32)

        @plsc.parallel_loop(0, GROUPS, 1, unroll=UNROLL_G, carry=(m0, b0))
        def scan_carry(g, carry):
            m, bb = carry
            base = g * GELEMS
            v = [buf[slot, pl.ds(base + j * L, L)] for j in range(GV)]
            while len(v) > 1:
                v = [jnp.maximum(v[a], v[a + 1]) for a in range(0, len(v), 2)]
            pred = v[0] > m
            m = jnp.where(pred, v[0], m)
            bb = jnp.where(pred, g, bb)
            return m, bb

        m, bb = scan_carry

        _, gstar = _tournament(
            [m[j] for j in range(L)], [bb[j] for j in range(L)]
        )

        gbase = gstar * GELEMS
        ivbase = gbase + lane
        fm = jnp.full((L,), NEG_INF, jnp.float32)
        fi = jnp.zeros((L,), jnp.int32)
        for j in range(GV):
            v = buf[slot, pl.ds(gbase + j * L, L)]
            pred = v > fm
            fm = jnp.where(pred, v, fm)
            fi = jnp.where(pred, ivbase + j * L, fi)
        _, ridx = _tournament([fm[j] for j in range(L)], [fi[j] for j in range(L)])
        res_vec = jnp.where(lane == r, ridx, res_vec)

    res[...] = res_vec
    pltpu.sync_copy(res, out_hbm.at[wid])


def _tc_body(x_ref, o_ref):
    x = x_ref[...]
    mx = jnp.max(x, axis=1, keepdims=True)
    idx = lax.broadcasted_iota(jnp.int32, x.shape, 1)
    cand = jnp.where(x == mx, idx, COLS)
    o_ref[...] = jnp.min(cand, axis=1, keepdims=True)


_argmax_tc = pl.pallas_call(
    _tc_body,
    grid=(TC_ROWS // TC_BR,),
    in_specs=[pl.BlockSpec((TC_BR, COLS), lambda i: (i, 0))],
    out_specs=pl.BlockSpec((TC_BR, 1), lambda i: (i, 0)),
    out_shape=jax.ShapeDtypeStruct((TC_ROWS, 1), jnp.int32),
)


def kernel(logits):
    tc_idx = _argmax_tc(logits)
    staged = _argmax_sc(logits)
    sc_idx = staged[:, :ROWS_PER_W].reshape(SC_ROWS, 1)
    return jnp.concatenate([tc_idx, sc_idx], axis=0)
